# baseline (speedup 1.0000x reference)
import jax
import jax.numpy as jnp
from jax.experimental import pallas as pl
from jax.experimental.pallas import tpu as pltpu

_NBUF = 8
_GROUP = 1
_OBUF = 4


def _moe_manual_kernel(bias_ref, x_hbm, w_hbm, o_hbm, xbuf, wbuf, obuf, isem, osem):
    n_chunks = x_hbm.shape[0] // _GROUP

    def in_copies(c, slot):
        xc = pltpu.make_async_copy(
            x_hbm.at[pl.ds(c * _GROUP, _GROUP)], xbuf.at[slot], isem.at[slot, 0]
        )
        wc = pltpu.make_async_copy(
            w_hbm.at[pl.ds(c * _GROUP, _GROUP)], wbuf.at[slot], isem.at[slot, 1]
        )
        return xc, wc

    def out_copy(c, oslot):
        return pltpu.make_async_copy(
            obuf.at[oslot], o_hbm.at[pl.ds(c * _GROUP, _GROUP)], osem.at[oslot]
        )

    for s in range(_NBUF):
        xc, wc = in_copies(s, s)
        xc.start()
        wc.start()

    bias = (bias_ref[0] - x_hbm.shape[1]).astype(jnp.float32)

    def body(c, carry):
        slot = jax.lax.rem(c, _NBUF)
        oslot = jax.lax.rem(c, _OBUF)
        xc, wc = in_copies(c, slot)
        xc.wait()
        wc.wait()

        @pl.when(c >= _OBUF)
        def _():
            out_copy(c - _OBUF, oslot).wait()

        for i in range(_GROUP):
            x = xbuf[slot, i].astype(jnp.bfloat16)
            w = wbuf[slot, i].astype(jnp.bfloat16)
            acc = jax.lax.dot_general(
                x, w, (((1,), (1,)), ((), ())), preferred_element_type=jnp.float32
            )
            obuf[oslot, i] = acc + bias

        out_copy(c, oslot).start()

        @pl.when(c + _NBUF < n_chunks)
        def _():
            xc2, wc2 = in_copies(c + _NBUF, slot)
            xc2.start()
            wc2.start()

        return carry

    jax.lax.fori_loop(0, n_chunks, body, 0)
    for c in range(max(n_chunks - _OBUF, 0), n_chunks):
        out_copy(c, c % _OBUF).wait()


def kernel(inputs, moe_weight, expert_size):
    num_experts, output_size, input_size = moe_weight.shape
    total_tokens = inputs.shape[0]
    tokens_per_expert = total_tokens // num_experts

    x = inputs.reshape(num_experts, tokens_per_expert, input_size)
    es = jnp.asarray(expert_size, jnp.int32).reshape(1)

    out = pl.pallas_call(
        _moe_manual_kernel,
        in_specs=[
            pl.BlockSpec(memory_space=pltpu.SMEM),
            pl.BlockSpec(memory_space=pltpu.HBM),
            pl.BlockSpec(memory_space=pltpu.HBM),
        ],
        out_specs=pl.BlockSpec(memory_space=pltpu.HBM),
        out_shape=jax.ShapeDtypeStruct(
            (num_experts, tokens_per_expert, output_size), jnp.float32
        ),
        scratch_shapes=[
            pltpu.VMEM((_NBUF, _GROUP, tokens_per_expert, input_size), jnp.float32),
            pltpu.VMEM((_NBUF, _GROUP, output_size, input_size), jnp.float32),
            pltpu.VMEM((_OBUF, _GROUP, tokens_per_expert, output_size), jnp.float32),
            pltpu.SemaphoreType.DMA((_NBUF, 2)),
            pltpu.SemaphoreType.DMA((_OBUF,)),
        ],
    )(es, x, moe_weight)
    return out.reshape(total_tokens, output_size)
